# Initial kernel scaffold; baseline (speedup 1.0000x reference)
import functools

import jax
import jax.numpy as jnp
from jax import lax
from jax.experimental import pallas as pl
from jax.experimental.pallas import tpu as pltpu
from jax.experimental.pallas import tpu_sc as plsc

N_NODES = 10000
D_FEAT = 128
N_EDGES = 320000

NC = 2
NS = 16
NW = NC * NS
CK = 128
NCH = (N_EDGES + NW * CK - 1) // (NW * CK)
E_PAD = NW * NCH * CK
ROWS_PER_TILE = N_NODES // NS


def _layer_body(table, srcs, dsts, ews, zeros, part,
                acc, src_v, dst_v, ew_v, rows_v, sem):
    c = lax.axis_index("c")
    s = lax.axis_index("s")
    wid = c * NS + s

    pltpu.sync_copy(srcs.at[wid], src_v)
    pltpu.sync_copy(dsts.at[wid], dst_v)
    pltpu.sync_copy(ews.at[wid], ew_v)

    pltpu.sync_copy(zeros, acc.at[pl.ds(s * ROWS_PER_TILE, ROWS_PER_TILE)])
    plsc.subcore_barrier()

    def chunk(j, carry):
        pltpu.async_copy(table.at[src_v.at[j]], rows_v, sem).wait()

        def rowgrp(rb, carry2):
            for i in range(16):
                r = rb * 16 + i
                w = plsc.load_gather(
                    ew_v,
                    [jnp.full((16,), j, jnp.int32),
                     jnp.full((16,), r, jnp.int32)])
                for cb in range(8):
                    sl = rows_v[r, pl.ds(cb * 16, 16)]
                    rows_v[r, pl.ds(cb * 16, 16)] = sl * w
            return carry2

        lax.fori_loop(0, 8, rowgrp, 0)

        pltpu.sync_copy(rows_v, acc.at[dst_v.at[j]], add=True)
        return carry

    lax.fori_loop(0, NCH, chunk, 0)
    plsc.subcore_barrier()

    pltpu.sync_copy(acc.at[pl.ds(s * ROWS_PER_TILE, ROWS_PER_TILE)],
                    part.at[c, pl.ds(s * ROWS_PER_TILE, ROWS_PER_TILE)])


_layer = pl.kernel(
    _layer_body,
    out_type=jax.ShapeDtypeStruct((NC, N_NODES, D_FEAT), jnp.float32),
    mesh=plsc.VectorSubcoreMesh(core_axis_name="c", subcore_axis_name="s"),
    scratch_types=[
        pltpu.VMEM_SHARED((N_NODES, D_FEAT), jnp.float32),
        pltpu.VMEM((NCH, CK), jnp.int32),
        pltpu.VMEM((NCH, CK), jnp.int32),
        pltpu.VMEM((NCH, CK), jnp.float32),
        pltpu.VMEM((CK, D_FEAT), jnp.float32),
        pltpu.SemaphoreType.DMA,
    ],
)


def _combine_body(p_ref, o_ref):
    o_ref[...] = p_ref[0] + p_ref[1]


def _final_body(x_ref, h1_ref, p_ref, o_ref):
    o_ref[...] = (x_ref[...] + h1_ref[...] + p_ref[0] + p_ref[1]) * (1.0 / 3.0)


_GB = 1250

_combine = pl.pallas_call(
    _combine_body,
    grid=(N_NODES // _GB,),
    in_specs=[pl.BlockSpec((NC, _GB, D_FEAT), lambda i: (0, i, 0))],
    out_specs=pl.BlockSpec((_GB, D_FEAT), lambda i: (i, 0)),
    out_shape=jax.ShapeDtypeStruct((N_NODES, D_FEAT), jnp.float32),
)

_final = pl.pallas_call(
    _final_body,
    grid=(N_NODES // _GB,),
    in_specs=[
        pl.BlockSpec((_GB, D_FEAT), lambda i: (i, 0)),
        pl.BlockSpec((_GB, D_FEAT), lambda i: (i, 0)),
        pl.BlockSpec((NC, _GB, D_FEAT), lambda i: (0, i, 0)),
    ],
    out_specs=pl.BlockSpec((_GB, D_FEAT), lambda i: (i, 0)),
    out_shape=jax.ShapeDtypeStruct((N_NODES, D_FEAT), jnp.float32),
)


def kernel(x, edge_index, edge_weight):
    src = edge_index[0].astype(jnp.int32)
    dst = edge_index[1].astype(jnp.int32)
    ew = edge_weight.astype(jnp.float32)

    pad = E_PAD - N_EDGES
    src = jnp.pad(src, (0, pad)).reshape(NW, NCH, CK)
    dst = jnp.pad(dst, (0, pad)).reshape(NW, NCH, CK)
    ew = jnp.pad(ew, (0, pad)).reshape(NW, NCH, CK)

    zeros = jnp.zeros((ROWS_PER_TILE, D_FEAT), jnp.float32)

    p1 = _layer(x, src, dst, ew, zeros)
    h1 = _combine(p1)
    p2 = _layer(h1, src, dst, ew, zeros)
    return _final(x, h1, p2)


# speedup vs baseline: 3.7975x; 3.7975x over previous
import functools

import jax
import jax.numpy as jnp
from jax import lax
from jax.experimental import pallas as pl
from jax.experimental.pallas import tpu as pltpu
from jax.experimental.pallas import tpu_sc as plsc

N_NODES = 10000
D_FEAT = 128
N_EDGES = 320000

NC = 2
NS = 16
NW = NC * NS
CK = 128
NCH = (N_EDGES + NW * CK - 1) // (NW * CK)
E_PAD = NW * NCH * CK
N_PAD = 10240
ROWS_PER_TILE = N_PAD // NS


def _layer_body(table, srcs, dsts, ews, zeros, part,
                acc, src_v, dst_v, ew_v, rows_v, sem):
    c = lax.axis_index("c")
    s = lax.axis_index("s")
    wid = c * NS + s

    pltpu.sync_copy(srcs.at[wid], src_v)
    pltpu.sync_copy(dsts.at[wid], dst_v)
    pltpu.sync_copy(ews.at[wid], ew_v)

    pltpu.sync_copy(zeros, acc.at[pl.ds(s * ROWS_PER_TILE, ROWS_PER_TILE)])
    plsc.subcore_barrier()

    def chunk(j, carry):
        pltpu.async_copy(table.at[src_v.at[j]], rows_v, sem).wait()

        def rowgrp(rb, carry2):
            for i in range(16):
                r = rb * 16 + i
                w = plsc.load_gather(
                    ew_v, [jnp.full((16,), j * CK + r, jnp.int32)])
                for cb in range(8):
                    sl = rows_v[r, pl.ds(cb * 16, 16)]
                    rows_v[r, pl.ds(cb * 16, 16)] = sl * w
            return carry2

        lax.fori_loop(0, 8, rowgrp, 0)

        pltpu.sync_copy(rows_v, acc.at[dst_v.at[j]], add=True)
        return carry

    lax.fori_loop(0, NCH, chunk, 0)
    plsc.subcore_barrier()

    pltpu.sync_copy(acc.at[pl.ds(s * ROWS_PER_TILE, ROWS_PER_TILE)],
                    part.at[c, pl.ds(s * ROWS_PER_TILE, ROWS_PER_TILE)])


@functools.cache
def _get_layer():
    return pl.kernel(
        _layer_body,
        out_type=jax.ShapeDtypeStruct((NC, N_PAD, D_FEAT), jnp.float32),
        mesh=plsc.VectorSubcoreMesh(core_axis_name="c", subcore_axis_name="s",
                                    num_cores=NC, num_subcores=NS),
        compiler_params=pltpu.CompilerParams(needs_layout_passes=False),
        scratch_types=[
            pltpu.VMEM_SHARED((N_PAD, D_FEAT), jnp.float32),
            pltpu.VMEM((NCH, CK), jnp.int32),
            pltpu.VMEM((NCH, CK), jnp.int32),
            pltpu.VMEM((NCH * CK,), jnp.float32),
            pltpu.VMEM((CK, D_FEAT), jnp.float32),
            pltpu.SemaphoreType.DMA,
        ],
    )


def _combine_body(p_ref, o_ref):
    o_ref[...] = p_ref[0] + p_ref[1]


def _final_body(x_ref, h1_ref, p_ref, o_ref):
    o_ref[...] = (x_ref[...] + h1_ref[...] + p_ref[0] + p_ref[1]) * (1.0 / 3.0)


_GB = 1000

_combine = pl.pallas_call(
    _combine_body,
    grid=(N_NODES // _GB,),
    in_specs=[pl.BlockSpec((NC, _GB, D_FEAT), lambda i: (0, i, 0))],
    out_specs=pl.BlockSpec((_GB, D_FEAT), lambda i: (i, 0)),
    out_shape=jax.ShapeDtypeStruct((N_NODES, D_FEAT), jnp.float32),
)

_final = pl.pallas_call(
    _final_body,
    grid=(N_NODES // _GB,),
    in_specs=[
        pl.BlockSpec((_GB, D_FEAT), lambda i: (i, 0)),
        pl.BlockSpec((_GB, D_FEAT), lambda i: (i, 0)),
        pl.BlockSpec((NC, _GB, D_FEAT), lambda i: (0, i, 0)),
    ],
    out_specs=pl.BlockSpec((_GB, D_FEAT), lambda i: (i, 0)),
    out_shape=jax.ShapeDtypeStruct((N_NODES, D_FEAT), jnp.float32),
)


def kernel(x, edge_index, edge_weight):
    src = edge_index[0].astype(jnp.int32)
    dst = edge_index[1].astype(jnp.int32)
    ew = edge_weight.astype(jnp.float32)

    pad = E_PAD - N_EDGES
    src = jnp.pad(src, (0, pad)).reshape(NW, NCH, CK)
    dst = jnp.pad(dst, (0, pad)).reshape(NW, NCH, CK)
    ew = jnp.pad(ew, (0, pad)).reshape(NW, NCH * CK)

    zeros = jnp.zeros((ROWS_PER_TILE, D_FEAT), jnp.float32)

    layer = _get_layer()
    p1 = layer(x, src, dst, ew, zeros)
    h1 = _combine(p1)
    p2 = layer(h1, src, dst, ew, zeros)
    return _final(x, h1, p2)
